# baseline (speedup 1.0000x reference)
import functools
import jax
import jax.numpy as jnp
from jax import lax
from jax.experimental import pallas as pl
from jax.experimental.pallas import tpu as pltpu
from jax.experimental.pallas import tpu_sc as plsc

N = 10000
E = 320000
D = 128
M = 128
O = 128

NC = 2
NS = 16
L = 16
NW = NC * NS
EPW = E // NW
C1 = 2000
C2 = 80


@functools.lru_cache(maxsize=None)
def _get_mesh():
    return plsc.VectorSubcoreMesh(core_axis_name="c", subcore_axis_name="s",
                                  num_cores=NC, num_subcores=NS)


def _node_precompute_body(x_ref, wm1t, wm2t, wm3t, bm1, bm2, bm3, m_out):
    xb = x_ref[...]
    h = jnp.maximum(jnp.dot(xb, wm1t[...], preferred_element_type=jnp.float32) + bm1[...], 0.0)
    h = jnp.maximum(jnp.dot(h, wm2t[...], preferred_element_type=jnp.float32) + bm2[...], 0.0)
    h = jnp.maximum(jnp.dot(h, wm3t[...], preferred_element_type=jnp.float32) + bm3[...], 0.0)
    m_out[...] = h


def _node_precompute(x, wm1t, wm2t, wm3t, bm1, bm2, bm3):
    BN = 1000
    grid = N // BN
    return pl.pallas_call(
        _node_precompute_body,
        grid=(grid,),
        in_specs=[
            pl.BlockSpec((BN, D), lambda i: (i, 0)),
            pl.BlockSpec((D, M), lambda i: (0, 0)),
            pl.BlockSpec((M, M), lambda i: (0, 0)),
            pl.BlockSpec((M, M), lambda i: (0, 0)),
            pl.BlockSpec((1, M), lambda i: (0, 0)),
            pl.BlockSpec((1, M), lambda i: (0, 0)),
            pl.BlockSpec((1, M), lambda i: (0, 0)),
        ],
        out_specs=pl.BlockSpec((BN, M), lambda i: (i, 0)),
        out_shape=jax.ShapeDtypeStruct((N, M), jnp.float32),
    )(x, wm1t, wm2t, wm3t, bm1, bm2, bm3)


def _score_table_body(x_ref, w8, b8, stbl_out):
    stbl_out[...] = lax.dot_general(
        w8[...], x_ref[...], (((0,), (1,)), ((), ())),
        preferred_element_type=jnp.float32) + b8[...]


def _score_table(x, w8, b8):
    return pl.pallas_call(
        _score_table_body,
        out_shape=jax.ShapeDtypeStruct((8, N), jnp.float32),
    )(x, w8, b8)


R1 = 3
NCH1 = EPW // C1
NV1 = NCH1 + 1
assert NV1 % R1 == 0


def _sc_att_body(s1s_hbm, s2s_hbm, s3s_hbm, s1d_hbm, s2d_hbm, s3d_hbm,
                 src_hbm, dst_hbm,
                 a1_hbm, a2_hbm, a3_hbm, acc4_hbm,
                 is0, is1, is2, id0, id1, id2,
                 g10, g11, g12, g20, g21, g22, g30, g31, g32,
                 h10, h11, h12, h20, h21, h22, h30, h31, h32,
                 b10, b11, b12, b20, b21, b22, b30, b31, b32,
                 ones_v, zb,
                 sA0, sA1, sA2, sG0, sG1, sG2, sW0, sW1, sW2, sC0, sC1_, sC2_,
                 acc1_sh, acc2_sh, acc3_sh, accd_sh):
    idx_s = [is0, is1, is2]
    idx_d = [id0, id1, id2]
    gs = [[g10, g20, g30], [g11, g21, g31], [g12, g22, g32]]
    gd = [[h10, h20, h30], [h11, h21, h31], [h12, h22, h32]]
    ab = [[b10, b20, b30], [b11, b21, b31], [b12, b22, b32]]
    semA = [sA0, sA1, sA2]
    semG = [sG0, sG1, sG2]
    semW = [sW0, sW1, sW2]
    semC = [sC0, sC1_, sC2_]
    stables = [s1s_hbm, s2s_hbm, s3s_hbm]
    dtables = [s1d_hbm, s2d_hbm, s3d_hbm]
    atables = [a1_hbm, a2_hbm, a3_hbm]
    accs = [acc1_sh, acc2_sh, acc3_sh, accd_sh]
    c = lax.axis_index("c")
    s = lax.axis_index("s")
    wid = s * NC + c

    ones16 = jnp.full((L,), 1.0, jnp.float32)
    zero16 = jnp.zeros((L,), jnp.float32)

    @pl.loop(0, C1 // L)
    def _(g):
        ones_v[pl.ds(g * L, L)] = ones16
        zb[pl.ds(g * L, L)] = zero16

    @pl.when(s < N // C1)
    def _():
        for acc in accs:
            pltpu.sync_copy(zb, acc.at[pl.ds(s * C1, C1)])
    plsc.subcore_barrier()

    def baseof(j):
        return wid * EPW + j * C1

    def issue_stage_a(j, sl):
        b = baseof(j)
        pltpu.async_copy(src_hbm.at[pl.ds(b, C1)], idx_s[sl], semA[sl])
        pltpu.async_copy(dst_hbm.at[pl.ds(b, C1)], idx_d[sl], semA[sl])

    def wait_stage_a(j, sl):
        b = baseof(j)
        pltpu.make_async_copy(src_hbm.at[pl.ds(b, C1)], idx_s[sl], semA[sl]).wait()
        pltpu.make_async_copy(dst_hbm.at[pl.ds(b, C1)], idx_d[sl], semA[sl]).wait()

    def issue_gathers(sl):
        for i in range(3):
            pltpu.async_copy(stables[i].at[idx_s[sl]], gs[sl][i], semG[sl])
            pltpu.async_copy(dtables[i].at[idx_d[sl]], gd[sl][i], semG[sl])

    def wait_gathers(sl):
        for i in range(3):
            pltpu.make_async_copy(stables[i].at[idx_s[sl]], gs[sl][i], semG[sl]).wait()
            pltpu.make_async_copy(dtables[i].at[idx_d[sl]], gd[sl][i], semG[sl]).wait()

    def issue_outs(j, sl):
        b = baseof(j)
        for i in range(3):
            pltpu.async_copy(ab[sl][i], atables[i].at[pl.ds(b, C1)], semW[sl])
            pltpu.async_copy(ab[sl][i], accs[i].at[idx_d[sl]], semC[sl], add=True)
        pltpu.async_copy(ones_v, accd_sh.at[idx_d[sl]], semC[sl], add=True)

    def wait_outs(j, sl):
        b = baseof(j)
        for i in range(3):
            pltpu.make_async_copy(ab[sl][i], atables[i].at[pl.ds(b, C1)], semW[sl]).wait()
            pltpu.make_async_copy(ab[sl][i], accs[i].at[idx_d[sl]], semC[sl]).wait()
        pltpu.make_async_copy(ones_v, accd_sh.at[idx_d[sl]], semC[sl]).wait()

    issue_stage_a(0, 0)
    issue_stage_a(1, 1)
    wait_stage_a(0, 0)
    issue_gathers(0)

    @pl.loop(0, NV1 // R1)
    def _(t):
        for b in range(R1):
            k = t * R1 + b

            @pl.when(k + 1 < NCH1)
            def _():
                wait_stage_a(k + 1, (b + 1) % R1)
                issue_gathers((b + 1) % R1)

            @pl.when(k < NCH1)
            def _():
                wait_gathers(b)

                @pl.loop(0, C1 // L)
                def _(g):
                    sl = pl.ds(g * L, L)
                    for i in range(3):
                        ab[b][i][sl] = jnp.exp(
                            jnp.maximum(gs[b][i][sl] + gd[b][i][sl], 0.0))

                issue_outs(k, b)

            @pl.when(k >= 1)
            def _():
                wait_outs(k - 1, (b + 2) % R1)

            @pl.when(k + 2 < NCH1)
            def _():
                issue_stage_a(k + 2, (b + 2) % R1)

    plsc.subcore_barrier()

    @pl.when(s == 0)
    def _():
        for i in range(4):
            pltpu.sync_copy(accs[i], acc4_hbm.at[c, i])


@functools.lru_cache(maxsize=None)
def _sc_att():
    return pl.kernel(
        _sc_att_body,
        out_type=[
            jax.ShapeDtypeStruct((E,), jnp.float32),
            jax.ShapeDtypeStruct((E,), jnp.float32),
            jax.ShapeDtypeStruct((E,), jnp.float32),
            jax.ShapeDtypeStruct((NC, 4, N), jnp.float32),
        ],
        mesh=_get_mesh(),
        scratch_types=(
            [pltpu.VMEM((C1,), jnp.int32)] * 6
            + [pltpu.VMEM((C1,), jnp.float32)] * 18
            + [pltpu.VMEM((C1,), jnp.float32)] * 9
            + [pltpu.VMEM((C1,), jnp.float32)] * 2
            + [pltpu.SemaphoreType.DMA] * 12
            + [pltpu.VMEM_SHARED((N,), jnp.float32)] * 4
        ),
    )


def _recip_body(acc4_ref, rec_ref):
    a = acc4_ref[0] + acc4_ref[1]
    rec_ref[...] = jnp.where(a > 0.0, 1.0 / jnp.maximum(a, 1e-30), 0.0)


def _recip(acc4):
    return pl.pallas_call(
        _recip_body,
        out_shape=jax.ShapeDtypeStruct((4, N), jnp.float32),
    )(acc4)


R2_ = 2
NV2 = NCH1 + 1
assert NV2 % R2_ == 0


def _sc_w_body(a1_hbm, a2_hbm, a3_hbm, r1_hbm, r2_hbm, r3_hbm, rd_hbm,
               dst_hbm, w_hbm,
               jd0, jd1, p10, p11, p20, p21, p30, p31,
               q10, q11, q20, q21, q30, q31, q40, q41,
               wc0, wc1,
               sA0, sA1, sG0, sG1, sW0, sW1):
    idx_d = [jd0, jd1]
    ap = [[p10, p20, p30], [p11, p21, p31]]
    gr = [[q10, q20, q30, q40], [q11, q21, q31, q41]]
    wchunk = [wc0, wc1]
    semA = [sA0, sA1]
    semG = [sG0, sG1]
    semW = [sW0, sW1]
    atables = [a1_hbm, a2_hbm, a3_hbm]
    rtables = [r1_hbm, r2_hbm, r3_hbm, rd_hbm]
    c = lax.axis_index("c")
    s = lax.axis_index("s")
    wid = s * NC + c
    third = jnp.full((L,), 1.0 / 3.0, jnp.float32)

    def baseof(j):
        return wid * EPW + j * C1

    def issue_stage_a(j, sl):
        pltpu.async_copy(dst_hbm.at[pl.ds(baseof(j), C1)], idx_d[sl], semA[sl])

    def wait_stage_a(j, sl):
        pltpu.make_async_copy(dst_hbm.at[pl.ds(baseof(j), C1)], idx_d[sl],
                              semA[sl]).wait()

    def issue_loads(j, sl):
        b = baseof(j)
        for i in range(3):
            pltpu.async_copy(atables[i].at[pl.ds(b, C1)], ap[sl][i], semG[sl])
        for i in range(4):
            pltpu.async_copy(rtables[i].at[idx_d[sl]], gr[sl][i], semG[sl])

    def wait_loads(j, sl):
        b = baseof(j)
        for i in range(3):
            pltpu.make_async_copy(atables[i].at[pl.ds(b, C1)], ap[sl][i],
                                  semG[sl]).wait()
        for i in range(4):
            pltpu.make_async_copy(rtables[i].at[idx_d[sl]], gr[sl][i],
                                  semG[sl]).wait()

    def issue_wout(j, sl):
        pltpu.async_copy(wchunk[sl], w_hbm.at[pl.ds(baseof(j), C1)], semW[sl])

    def wait_wout(j, sl):
        pltpu.make_async_copy(wchunk[sl], w_hbm.at[pl.ds(baseof(j), C1)],
                              semW[sl]).wait()

    issue_stage_a(0, 0)
    issue_stage_a(1, 1)
    wait_stage_a(0, 0)
    issue_loads(0, 0)

    @pl.loop(0, NV2 // R2_)
    def _(t):
        for b in range(R2_):
            k = t * R2_ + b

            @pl.when(k + 1 < NCH1)
            def _():
                wait_stage_a(k + 1, (b + 1) % R2_)
                issue_loads(k + 1, (b + 1) % R2_)

            @pl.when(k < NCH1)
            def _():
                wait_loads(k, b)

                @pl.when(k >= 2)
                def _():
                    wait_wout(k - 2, b)

                @pl.loop(0, C1 // L)
                def _(g):
                    sl = pl.ds(g * L, L)
                    w = (ap[b][0][sl] * gr[b][0][sl]
                         + ap[b][1][sl] * gr[b][1][sl]
                         + ap[b][2][sl] * gr[b][2][sl])
                    wchunk[b][sl] = w * third * gr[b][3][sl]

                issue_wout(k, b)

            @pl.when(k + 2 < NCH1)
            def _():
                issue_stage_a(k + 2, (b + 2) % R2_)

    @pl.when(NCH1 >= 2)
    def _():
        wait_wout(NCH1 - 2, (NCH1 - 2) % R2_)
        wait_wout(NCH1 - 1, (NCH1 - 1) % R2_)


@functools.lru_cache(maxsize=None)
def _sc_w():
    return pl.kernel(
        _sc_w_body,
        out_type=jax.ShapeDtypeStruct((E,), jnp.float32),
        mesh=_get_mesh(),
        scratch_types=(
            [pltpu.VMEM((C1,), jnp.int32)] * 2
            + [pltpu.VMEM((C1,), jnp.float32)] * 6
            + [pltpu.VMEM((C1,), jnp.float32)] * 8
            + [pltpu.VMEM((C1,), jnp.float32)] * 2
            + [pltpu.SemaphoreType.DMA] * 6
        ),
    )


RING = 3
NCH = EPW // C2
NV = NCH + 1
assert NV % RING == 0


def _sc_agg_body(mnode_hbm, src_hbm, dst_hbm, w_hbm,
                 accM_hbm,
                 as0, as1, as2, ad0, ad1, ad2, aw0, aw1, aw2,
                 mr0, mr1, mr2,
                 sA0, sA1, sA2, sM0, sM1, sM2, sS0, sS1, sS2, accM_sh):
    asrc = [as0, as1, as2]
    adst = [ad0, ad1, ad2]
    aw = [aw0, aw1, aw2]
    mrows = [mr0, mr1, mr2]
    semA = [sA0, sA1, sA2]
    semM = [sM0, sM1, sM2]
    semS = [sS0, sS1, sS2]
    c = lax.axis_index("c")
    s = lax.axis_index("s")
    wid = s * NC + c

    zero16 = jnp.zeros((L,), jnp.float32)

    @pl.loop(0, C2)
    def _(r):
        for cg in range(M // L):
            mr0[r, pl.ds(cg * L, L)] = zero16

    @pl.loop(0, (N // C2 + NS - 1) // NS)
    def _(j):
        ch = j * NS + s

        @pl.when(ch < N // C2)
        def _():
            pltpu.sync_copy(mr0, accM_sh.at[pl.ds(ch * C2, C2)])

    plsc.subcore_barrier()

    def baseof(j):
        return wid * EPW + j * C2

    def issue_stage_a(j, sl):
        b = baseof(j)
        pltpu.async_copy(src_hbm.at[pl.ds(b, C2)], asrc[sl], semA[sl])
        pltpu.async_copy(dst_hbm.at[pl.ds(b, C2)], adst[sl], semA[sl])
        pltpu.async_copy(w_hbm.at[pl.ds(b, C2)], aw[sl], semA[sl])

    def wait_stage_a(j, sl):
        b = baseof(j)
        pltpu.make_async_copy(src_hbm.at[pl.ds(b, C2)], asrc[sl], semA[sl]).wait()
        pltpu.make_async_copy(dst_hbm.at[pl.ds(b, C2)], adst[sl], semA[sl]).wait()
        pltpu.make_async_copy(w_hbm.at[pl.ds(b, C2)], aw[sl], semA[sl]).wait()

    def issue_gather(sl):
        pltpu.async_copy(mnode_hbm.at[asrc[sl]], mrows[sl], semM[sl])

    def wait_gather(sl):
        pltpu.make_async_copy(mnode_hbm.at[asrc[sl]], mrows[sl], semM[sl]).wait()

    def issue_scatter(sl):
        pltpu.async_copy(mrows[sl], accM_sh.at[adst[sl]], semS[sl], add=True)

    def wait_scatter(sl):
        pltpu.make_async_copy(mrows[sl], accM_sh.at[adst[sl]], semS[sl]).wait()

    issue_stage_a(0, 0)
    issue_stage_a(1, 1)
    wait_stage_a(0, 0)
    issue_gather(0)

    @pl.loop(0, NV // RING)
    def _(t):
        for b in range(RING):
            k = t * RING + b

            @pl.when(k + 1 < NCH)
            def _():
                wait_stage_a(k + 1, (b + 1) % RING)
                issue_gather((b + 1) % RING)

            @pl.when(k < NCH)
            def _():
                wait_gather(b)

                @pl.loop(0, C2 // L)
                def _(g):
                    w16 = aw[b][pl.ds(g * L, L)]
                    for j in range(L):
                        wv = jnp.take_along_axis(
                            w16, jnp.full((L,), j, jnp.int32), axis=0)
                        r = g * L + j
                        for cg in range(M // L):
                            sl = pl.ds(cg * L, L)
                            mrows[b][r, sl] = mrows[b][r, sl] * wv

                issue_scatter(b)

            @pl.when(k >= 1)
            def _():
                wait_scatter((b + 2) % RING)

            @pl.when(k + 2 < NCH)
            def _():
                issue_stage_a(k + 2, (b + 2) % RING)

    plsc.subcore_barrier()

    @pl.when(s == 0)
    def _():
        pltpu.sync_copy(accM_sh, accM_hbm.at[c])


@functools.lru_cache(maxsize=None)
def _sc_agg():
    return pl.kernel(
        _sc_agg_body,
        out_type=jax.ShapeDtypeStruct((NC, N, M), jnp.float32),
        mesh=_get_mesh(),
        scratch_types=(
            [pltpu.VMEM((C2,), jnp.int32)] * 6
            + [pltpu.VMEM((C2,), jnp.float32)] * 3
            + [pltpu.VMEM((C2, M), jnp.float32)] * 3
            + [pltpu.SemaphoreType.DMA] * 9
            + [pltpu.VMEM_SHARED((N, M), jnp.float32)]
        ),
    )


def _combine_body(x_ref, accM_ref, wc1xt, wc1ht, bc1, wc2t, bc2, out_ref):
    hn = accM_ref[0] + accM_ref[1]
    t = jnp.maximum(
        jnp.dot(x_ref[...], wc1xt[...], preferred_element_type=jnp.float32)
        + jnp.dot(hn, wc1ht[...], preferred_element_type=jnp.float32)
        + bc1[...], 0.0)
    out_ref[...] = jnp.dot(t, wc2t[...], preferred_element_type=jnp.float32) + bc2[...]


def _combine(x, accM, wc1xt, wc1ht, bc1, wc2t, bc2):
    BN = 1000
    grid = N // BN
    return pl.pallas_call(
        _combine_body,
        grid=(grid,),
        in_specs=[
            pl.BlockSpec((BN, D), lambda i: (i, 0)),
            pl.BlockSpec((NC, BN, M), lambda i: (0, i, 0)),
            pl.BlockSpec((D, O), lambda i: (0, 0)),
            pl.BlockSpec((M, O), lambda i: (0, 0)),
            pl.BlockSpec((1, O), lambda i: (0, 0)),
            pl.BlockSpec((O, O), lambda i: (0, 0)),
            pl.BlockSpec((1, O), lambda i: (0, 0)),
        ],
        out_specs=pl.BlockSpec((BN, O), lambda i: (i, 0)),
        out_shape=jax.ShapeDtypeStruct((N, O), jnp.float32),
    )(x, accM, wc1xt, wc1ht, bc1, wc2t, bc2)


@jax.jit
def kernel(x, edge_index, Wm1, bm1, Wm2, bm2, Wm3, bm3,
           Wa1, ba1, Wa2, ba2, Wa3, ba3, Wc1, bc1, Wc2, bc2):
    src = edge_index[0]
    dst = edge_index[1]

    zc = jnp.zeros((D,), jnp.float32)
    w8 = jnp.stack([Wa1[0, :D], Wa2[0, :D], Wa3[0, :D],
                    Wa1[0, D:], Wa2[0, D:], Wa3[0, D:], zc, zc], axis=1)
    b8 = jnp.concatenate([ba1, ba2, ba3, jnp.zeros((5,), jnp.float32)]).reshape(8, 1)

    m_node = _node_precompute(
        x, Wm1.T, Wm2.T, Wm3.T,
        bm1.reshape(1, M), bm2.reshape(1, M), bm3.reshape(1, M))
    stbl = _score_table(x, w8, b8)

    a1, a2, a3, acc4 = _sc_att()(
        stbl[0], stbl[1], stbl[2], stbl[3], stbl[4], stbl[5], src, dst)

    rec = _recip(acc4)

    w = _sc_w()(a1, a2, a3, rec[0], rec[1], rec[2], rec[3], dst)

    accM = _sc_agg()(m_node, src, dst, w)

    return _combine(x, accM,
                    Wc1[:, :D].T, Wc1[:, D:].T, bc1.reshape(1, O),
                    Wc2.T, bc2.reshape(1, O))


# speedup vs baseline: 1.0002x; 1.0002x over previous
import functools
import jax
import jax.numpy as jnp
from jax import lax
from jax.experimental import pallas as pl
from jax.experimental.pallas import tpu as pltpu
from jax.experimental.pallas import tpu_sc as plsc

N = 10000
E = 320000
D = 128
M = 128
O = 128

NC = 2
NS = 16
L = 16
NW = NC * NS
EPW = E // NW
C1 = 2000
C2 = 80

@functools.lru_cache(maxsize=None)
def _get_mesh():
    return plsc.VectorSubcoreMesh(core_axis_name="c", subcore_axis_name="s",
                                  num_cores=NC, num_subcores=NS)


def _node_precompute_body(x_ref, wm1t, wm2t, wm3t, bm1, bm2, bm3,
                          wsrc, wdst, bsrc, m_out, ssrc_out, sdst_out):
    xb = x_ref[...]
    h = jnp.maximum(jnp.dot(xb, wm1t[...], preferred_element_type=jnp.float32) + bm1[...], 0.0)
    h = jnp.maximum(jnp.dot(h, wm2t[...], preferred_element_type=jnp.float32) + bm2[...], 0.0)
    h = jnp.maximum(jnp.dot(h, wm3t[...], preferred_element_type=jnp.float32) + bm3[...], 0.0)
    m_out[...] = h
    ssrc_out[...] = jnp.dot(xb, wsrc[...], preferred_element_type=jnp.float32) + bsrc[...]
    sdst_out[...] = jnp.dot(xb, wdst[...], preferred_element_type=jnp.float32)


def _node_precompute(x, wm1t, wm2t, wm3t, bm1, bm2, bm3, wsrc, wdst, bsrc):
    BN = 1000
    grid = N // BN
    return pl.pallas_call(
        _node_precompute_body,
        grid=(grid,),
        in_specs=[
            pl.BlockSpec((BN, D), lambda i: (i, 0)),
            pl.BlockSpec((D, M), lambda i: (0, 0)),
            pl.BlockSpec((M, M), lambda i: (0, 0)),
            pl.BlockSpec((M, M), lambda i: (0, 0)),
            pl.BlockSpec((1, M), lambda i: (0, 0)),
            pl.BlockSpec((1, M), lambda i: (0, 0)),
            pl.BlockSpec((1, M), lambda i: (0, 0)),
            pl.BlockSpec((D, 4), lambda i: (0, 0)),
            pl.BlockSpec((D, 4), lambda i: (0, 0)),
            pl.BlockSpec((1, 4), lambda i: (0, 0)),
        ],
        out_specs=[
            pl.BlockSpec((BN, M), lambda i: (i, 0)),
            pl.BlockSpec((BN, 4), lambda i: (i, 0)),
            pl.BlockSpec((BN, 4), lambda i: (i, 0)),
        ],
        out_shape=[
            jax.ShapeDtypeStruct((N, M), jnp.float32),
            jax.ShapeDtypeStruct((N, 4), jnp.float32),
            jax.ShapeDtypeStruct((N, 4), jnp.float32),
        ],
    )(x, wm1t, wm2t, wm3t, bm1, bm2, bm3, wsrc, wdst, bsrc)


def _sc_att_body(ssrc_hbm, sdst_hbm, src_hbm, dst_hbm, z4_hbm,
                 ae_hbm, acc4_hbm,
                 idx_s, idx_d, idx4s, idx4d, idxsc, gs, gd, arows,
                 sem_a, sem_b, acc4_sh):
    c = lax.axis_index("c")
    s = lax.axis_index("s")
    wid = s * NC + c

    @pl.when(s == 0)
    def _():
        pltpu.sync_copy(z4_hbm, acc4_sh)
    plsc.subcore_barrier()

    ones16 = jnp.full((L,), 1.0, jnp.float32)

    @pl.loop(0, C1 // L)
    def _(g):
        arows[pl.ds(3 * C1 + g * L, L)] = ones16

    @pl.loop(0, EPW // C1)
    def _(k):
        base = wid * EPW + k * C1
        pltpu.sync_copy(src_hbm.at[pl.ds(base, C1)], idx_s)
        pltpu.sync_copy(dst_hbm.at[pl.ds(base, C1)], idx_d)

        @pl.loop(0, C1 // L)
        def _(g):
            sl = pl.ds(g * L, L)
            sv = idx_s[sl] * 4
            dv = idx_d[sl] * 4
            for i in range(3):
                idx4s[pl.ds(i * C1 + g * L, L)] = sv + i
                idx4d[pl.ds(i * C1 + g * L, L)] = dv + i
                idxsc[pl.ds(i * C1 + g * L, L)] = dv + i
            idxsc[pl.ds(3 * C1 + g * L, L)] = dv + 3

        cp_a = pltpu.async_copy(ssrc_hbm.at[idx4s], gs, sem_a)
        cp_b = pltpu.async_copy(sdst_hbm.at[idx4d], gd, sem_b)
        cp_a.wait()
        cp_b.wait()

        @pl.loop(0, 3 * C1 // L)
        def _(g):
            sl = pl.ds(g * L, L)
            arows[sl] = jnp.exp(jnp.maximum(gs[sl] + gd[sl], 0.0))

        pltpu.sync_copy(arows.at[pl.ds(0, 3 * C1)], ae_hbm.at[pl.ds(3 * base, 3 * C1)])
        pltpu.sync_copy(arows, acc4_sh.at[idxsc], add=True)

    plsc.subcore_barrier()

    @pl.when(s == 0)
    def _():
        pltpu.sync_copy(acc4_sh, acc4_hbm.at[c])


@functools.lru_cache(maxsize=None)
def _sc_att():
    return pl.kernel(
        _sc_att_body,
        out_type=[
            jax.ShapeDtypeStruct((3 * E,), jnp.float32),
            jax.ShapeDtypeStruct((NC, 4 * N), jnp.float32),
        ],
        mesh=_get_mesh(),
        scratch_types=[
            pltpu.VMEM((C1,), jnp.int32),
            pltpu.VMEM((C1,), jnp.int32),
            pltpu.VMEM((3 * C1,), jnp.int32),
            pltpu.VMEM((3 * C1,), jnp.int32),
            pltpu.VMEM((4 * C1,), jnp.int32),
            pltpu.VMEM((3 * C1,), jnp.float32),
            pltpu.VMEM((3 * C1,), jnp.float32),
            pltpu.VMEM((4 * C1,), jnp.float32),
            pltpu.SemaphoreType.DMA,
            pltpu.SemaphoreType.DMA,
            pltpu.VMEM_SHARED((4 * N,), jnp.float32),
        ],
    )


def _recip_body(acc4_ref, rec_ref):
    a = acc4_ref[0] + acc4_ref[1]
    rec_ref[...] = jnp.where(a > 0.0, 1.0 / jnp.maximum(a, 1e-30), 0.0)


def _recip(acc4):
    return pl.pallas_call(
        _recip_body,
        out_shape=jax.ShapeDtypeStruct((N, 4), jnp.float32),
    )(acc4)


def _sc_w_body(ae_hbm, rec_hbm, dst_hbm, w_hbm,
               idx_d, idx4r, ap, gr, wchunk, sem_a):
    c = lax.axis_index("c")
    s = lax.axis_index("s")
    wid = s * NC + c
    third = jnp.full((L,), 1.0 / 3.0, jnp.float32)

    @pl.loop(0, EPW // C1)
    def _(k):
        base = wid * EPW + k * C1
        pltpu.sync_copy(dst_hbm.at[pl.ds(base, C1)], idx_d)
        for i in range(3):
            pltpu.sync_copy(ae_hbm.at[pl.ds(3 * base + i * C1, C1)],
                            ap.at[pl.ds(i * C1, C1)])

        @pl.loop(0, C1 // L)
        def _(g):
            sl = pl.ds(g * L, L)
            dv = idx_d[sl] * 4
            for i in range(3):
                idx4r[pl.ds(i * C1 + g * L, L)] = dv + i

        pltpu.async_copy(rec_hbm.at[idx4r], gr, sem_a).wait()

        @pl.loop(0, C1 // L)
        def _(g):
            sl0 = pl.ds(g * L, L)
            sl1 = pl.ds(C1 + g * L, L)
            sl2 = pl.ds(2 * C1 + g * L, L)
            w = ap[sl0] * gr[sl0] + ap[sl1] * gr[sl1] + ap[sl2] * gr[sl2]
            wchunk[pl.ds(g * L, L)] = w * third

        pltpu.sync_copy(wchunk, w_hbm.at[pl.ds(base, C1)])


@functools.lru_cache(maxsize=None)
def _sc_w():
    return pl.kernel(
        _sc_w_body,
        out_type=jax.ShapeDtypeStruct((E,), jnp.float32),
        mesh=_get_mesh(),
        scratch_types=[
            pltpu.VMEM((C1,), jnp.int32),
            pltpu.VMEM((3 * C1,), jnp.int32),
            pltpu.VMEM((3 * C1,), jnp.float32),
            pltpu.VMEM((3 * C1,), jnp.float32),
            pltpu.VMEM((C1,), jnp.float32),
            pltpu.SemaphoreType.DMA,
        ],
    )


RING = 3
NCH = EPW // C2
NV = NCH + 1
assert NV % RING == 0


def _sc_agg_body(mnode_hbm, src_hbm, dst_hbm, w_hbm, zM_hbm,
                 accM_hbm,
                 as0, as1, as2, ad0, ad1, ad2, aw0, aw1, aw2,
                 mr0, mr1, mr2,
                 sA0, sA1, sA2, sM0, sM1, sM2, sS0, sS1, sS2, accM_sh):
    asrc = [as0, as1, as2]
    adst = [ad0, ad1, ad2]
    aw = [aw0, aw1, aw2]
    mrows = [mr0, mr1, mr2]
    semA = [sA0, sA1, sA2]
    semM = [sM0, sM1, sM2]
    semS = [sS0, sS1, sS2]
    c = lax.axis_index("c")
    s = lax.axis_index("s")
    wid = s * NC + c

    @pl.when(s == 0)
    def _():
        pltpu.sync_copy(zM_hbm, accM_sh)
    plsc.subcore_barrier()

    def baseof(j):
        return wid * EPW + j * C2

    def issue_stage_a(j, sl):
        b = baseof(j)
        pltpu.async_copy(src_hbm.at[pl.ds(b, C2)], asrc[sl], semA[sl])
        pltpu.async_copy(dst_hbm.at[pl.ds(b, C2)], adst[sl], semA[sl])
        pltpu.async_copy(w_hbm.at[pl.ds(b, C2)], aw[sl], semA[sl])

    def wait_stage_a(j, sl):
        b = baseof(j)
        pltpu.make_async_copy(src_hbm.at[pl.ds(b, C2)], asrc[sl], semA[sl]).wait()
        pltpu.make_async_copy(dst_hbm.at[pl.ds(b, C2)], adst[sl], semA[sl]).wait()
        pltpu.make_async_copy(w_hbm.at[pl.ds(b, C2)], aw[sl], semA[sl]).wait()

    def issue_gather(sl):
        pltpu.async_copy(mnode_hbm.at[asrc[sl]], mrows[sl], semM[sl])

    def wait_gather(sl):
        pltpu.make_async_copy(mnode_hbm.at[asrc[sl]], mrows[sl], semM[sl]).wait()

    def issue_scatter(sl):
        pltpu.async_copy(mrows[sl], accM_sh.at[adst[sl]], semS[sl], add=True)

    def wait_scatter(sl):
        pltpu.make_async_copy(mrows[sl], accM_sh.at[adst[sl]], semS[sl]).wait()

    issue_stage_a(0, 0)
    issue_stage_a(1, 1)
    wait_stage_a(0, 0)
    issue_gather(0)

    @pl.loop(0, NV // RING)
    def _(t):
        for b in range(RING):
            k = t * RING + b

            @pl.when(k + 1 < NCH)
            def _():
                wait_stage_a(k + 1, (b + 1) % RING)
                issue_gather((b + 1) % RING)

            @pl.when(k < NCH)
            def _():
                wait_gather(b)

                @pl.loop(0, C2 // L)
                def _(g):
                    w16 = aw[b][pl.ds(g * L, L)]
                    for j in range(L):
                        wv = jnp.take_along_axis(
                            w16, jnp.full((L,), j, jnp.int32), axis=0)
                        r = g * L + j
                        for cg in range(M // L):
                            sl = pl.ds(cg * L, L)
                            mrows[b][r, sl] = mrows[b][r, sl] * wv

                issue_scatter(b)

            @pl.when(k >= 1)
            def _():
                wait_scatter((b + 2) % RING)

            @pl.when(k + 2 < NCH)
            def _():
                issue_stage_a(k + 2, (b + 2) % RING)

    plsc.subcore_barrier()

    @pl.when(s == 0)
    def _():
        pltpu.sync_copy(accM_sh, accM_hbm.at[c])


@functools.lru_cache(maxsize=None)
def _sc_agg():
    return pl.kernel(
        _sc_agg_body,
        out_type=jax.ShapeDtypeStruct((NC, N, M), jnp.float32),
        mesh=_get_mesh(),
        scratch_types=(
            [pltpu.VMEM((C2,), jnp.int32)] * 6
            + [pltpu.VMEM((C2,), jnp.float32)] * 3
            + [pltpu.VMEM((C2, M), jnp.float32)] * 3
            + [pltpu.SemaphoreType.DMA] * 9
            + [pltpu.VMEM_SHARED((N, M), jnp.float32)]
        ),
    )


def _combine_body(x_ref, accM_ref, rec_ref, wc1xt, wc1ht, bc1, wc2t, bc2, out_ref):
    sum_m = accM_ref[0] + accM_ref[1]
    hn = sum_m * rec_ref[...][:, 3:4]
    t = jnp.maximum(
        jnp.dot(x_ref[...], wc1xt[...], preferred_element_type=jnp.float32)
        + jnp.dot(hn, wc1ht[...], preferred_element_type=jnp.float32)
        + bc1[...], 0.0)
    out_ref[...] = jnp.dot(t, wc2t[...], preferred_element_type=jnp.float32) + bc2[...]


def _combine(x, accM, rec, wc1xt, wc1ht, bc1, wc2t, bc2):
    BN = 1000
    grid = N // BN
    return pl.pallas_call(
        _combine_body,
        grid=(grid,),
        in_specs=[
            pl.BlockSpec((BN, D), lambda i: (i, 0)),
            pl.BlockSpec((NC, BN, M), lambda i: (0, i, 0)),
            pl.BlockSpec((BN, 4), lambda i: (i, 0)),
            pl.BlockSpec((D, O), lambda i: (0, 0)),
            pl.BlockSpec((M, O), lambda i: (0, 0)),
            pl.BlockSpec((1, O), lambda i: (0, 0)),
            pl.BlockSpec((O, O), lambda i: (0, 0)),
            pl.BlockSpec((1, O), lambda i: (0, 0)),
        ],
        out_specs=pl.BlockSpec((BN, O), lambda i: (i, 0)),
        out_shape=jax.ShapeDtypeStruct((N, O), jnp.float32),
    )(x, accM, rec, wc1xt, wc1ht, bc1, wc2t, bc2)


@jax.jit
def kernel(x, edge_index, Wm1, bm1, Wm2, bm2, Wm3, bm3,
           Wa1, ba1, Wa2, ba2, Wa3, ba3, Wc1, bc1, Wc2, bc2):
    src = edge_index[0]
    dst = edge_index[1]

    zcol = jnp.zeros((D, 1), jnp.float32)
    wsrc = jnp.concatenate([Wa1[:, :D].T, Wa2[:, :D].T, Wa3[:, :D].T, zcol], axis=1)
    wdst = jnp.concatenate([Wa1[:, D:].T, Wa2[:, D:].T, Wa3[:, D:].T, zcol], axis=1)
    bsrc = jnp.concatenate([ba1, ba2, ba3, jnp.zeros((1,), jnp.float32)]).reshape(1, 4)

    m_node, ssrc, sdst = _node_precompute(
        x, Wm1.T, Wm2.T, Wm3.T,
        bm1.reshape(1, M), bm2.reshape(1, M), bm3.reshape(1, M),
        wsrc, wdst, bsrc)

    z4 = jnp.zeros((4 * N,), jnp.float32)
    ae, acc4 = _sc_att()(ssrc.reshape(4 * N), sdst.reshape(4 * N), src, dst, z4)

    rec = _recip(acc4.reshape(NC, N, 4))

    w = _sc_w()(ae, rec.reshape(4 * N), dst)

    zM = jnp.zeros((N, M), jnp.float32)
    accM = _sc_agg()(m_node, src, dst, w, zM)

    return _combine(x, accM, rec,
                    Wc1[:, :D].T, Wc1[:, D:].T, bc1.reshape(1, O),
                    Wc2.T, bc2.reshape(1, O))


# speedup vs baseline: 1.1346x; 1.1344x over previous
import functools
import jax
import jax.numpy as jnp
from jax import lax
from jax.experimental import pallas as pl
from jax.experimental.pallas import tpu as pltpu
from jax.experimental.pallas import tpu_sc as plsc

N = 10000
E = 320000
D = 128
M = 128
O = 128

NC = 2
NS = 16
L = 16
NW = NC * NS
EPW = E // NW
C1 = 2000
C2 = 80

@functools.lru_cache(maxsize=None)
def _get_mesh():
    return plsc.VectorSubcoreMesh(core_axis_name="c", subcore_axis_name="s",
                                  num_cores=NC, num_subcores=NS)


def _node_precompute_body(x_ref, wm1t, wm2t, wm3t, bm1, bm2, bm3,
                          wsrc, wdst, bsrc, m_out, ssrc_out, sdst_out):
    xb = x_ref[...]
    h = jnp.maximum(jnp.dot(xb, wm1t[...], preferred_element_type=jnp.float32) + bm1[...], 0.0)
    h = jnp.maximum(jnp.dot(h, wm2t[...], preferred_element_type=jnp.float32) + bm2[...], 0.0)
    h = jnp.maximum(jnp.dot(h, wm3t[...], preferred_element_type=jnp.float32) + bm3[...], 0.0)
    m_out[...] = h
    ssrc_out[...] = jnp.dot(xb, wsrc[...], preferred_element_type=jnp.float32) + bsrc[...]
    sdst_out[...] = jnp.dot(xb, wdst[...], preferred_element_type=jnp.float32)


def _node_precompute(x, wm1t, wm2t, wm3t, bm1, bm2, bm3, wsrc, wdst, bsrc):
    BN = 1000
    grid = N // BN
    return pl.pallas_call(
        _node_precompute_body,
        grid=(grid,),
        in_specs=[
            pl.BlockSpec((BN, D), lambda i: (i, 0)),
            pl.BlockSpec((D, M), lambda i: (0, 0)),
            pl.BlockSpec((M, M), lambda i: (0, 0)),
            pl.BlockSpec((M, M), lambda i: (0, 0)),
            pl.BlockSpec((1, M), lambda i: (0, 0)),
            pl.BlockSpec((1, M), lambda i: (0, 0)),
            pl.BlockSpec((1, M), lambda i: (0, 0)),
            pl.BlockSpec((D, 4), lambda i: (0, 0)),
            pl.BlockSpec((D, 4), lambda i: (0, 0)),
            pl.BlockSpec((1, 4), lambda i: (0, 0)),
        ],
        out_specs=[
            pl.BlockSpec((BN, M), lambda i: (i, 0)),
            pl.BlockSpec((BN, 4), lambda i: (i, 0)),
            pl.BlockSpec((BN, 4), lambda i: (i, 0)),
        ],
        out_shape=[
            jax.ShapeDtypeStruct((N, M), jnp.float32),
            jax.ShapeDtypeStruct((N, 4), jnp.float32),
            jax.ShapeDtypeStruct((N, 4), jnp.float32),
        ],
    )(x, wm1t, wm2t, wm3t, bm1, bm2, bm3, wsrc, wdst, bsrc)


def _sc_att_body(ssrc_hbm, sdst_hbm, src_hbm, dst_hbm, z4_hbm,
                 ae_hbm, acc4_hbm,
                 idx_s, idx_d, idx4s, idx4d, idxsc, gs, gd, arows,
                 sem_a, sem_b, acc4_sh):
    c = lax.axis_index("c")
    s = lax.axis_index("s")
    wid = s * NC + c

    @pl.when(s == 0)
    def _():
        pltpu.sync_copy(z4_hbm, acc4_sh)
    plsc.subcore_barrier()

    ones16 = jnp.full((L,), 1.0, jnp.float32)

    @pl.loop(0, C1 // L)
    def _(g):
        arows[pl.ds(3 * C1 + g * L, L)] = ones16

    @pl.loop(0, EPW // C1)
    def _(k):
        base = wid * EPW + k * C1
        pltpu.sync_copy(src_hbm.at[pl.ds(base, C1)], idx_s)
        pltpu.sync_copy(dst_hbm.at[pl.ds(base, C1)], idx_d)

        @pl.loop(0, C1 // L)
        def _(g):
            sl = pl.ds(g * L, L)
            sv = idx_s[sl] * 4
            dv = idx_d[sl] * 4
            for i in range(3):
                idx4s[pl.ds(i * C1 + g * L, L)] = sv + i
                idx4d[pl.ds(i * C1 + g * L, L)] = dv + i
                idxsc[pl.ds(i * C1 + g * L, L)] = dv + i
            idxsc[pl.ds(3 * C1 + g * L, L)] = dv + 3

        cp_a = pltpu.async_copy(ssrc_hbm.at[idx4s], gs, sem_a)
        cp_b = pltpu.async_copy(sdst_hbm.at[idx4d], gd, sem_b)
        cp_a.wait()
        cp_b.wait()

        @pl.loop(0, 3 * C1 // L)
        def _(g):
            sl = pl.ds(g * L, L)
            arows[sl] = jnp.exp(jnp.maximum(gs[sl] + gd[sl], 0.0))

        for i in range(3):
            pltpu.sync_copy(arows.at[pl.ds(i * C1, C1)],
                            ae_hbm.at[pl.ds(i * E + base, C1)])
        pltpu.sync_copy(arows, acc4_sh.at[idxsc], add=True)

    plsc.subcore_barrier()

    @pl.when(s == 0)
    def _():
        pltpu.sync_copy(acc4_sh, acc4_hbm.at[c])


@functools.lru_cache(maxsize=None)
def _sc_att():
    return pl.kernel(
        _sc_att_body,
        out_type=[
            jax.ShapeDtypeStruct((3 * E,), jnp.float32),
            jax.ShapeDtypeStruct((NC, 4 * N), jnp.float32),
        ],
        mesh=_get_mesh(),
        scratch_types=[
            pltpu.VMEM((C1,), jnp.int32),
            pltpu.VMEM((C1,), jnp.int32),
            pltpu.VMEM((3 * C1,), jnp.int32),
            pltpu.VMEM((3 * C1,), jnp.int32),
            pltpu.VMEM((4 * C1,), jnp.int32),
            pltpu.VMEM((3 * C1,), jnp.float32),
            pltpu.VMEM((3 * C1,), jnp.float32),
            pltpu.VMEM((4 * C1,), jnp.float32),
            pltpu.SemaphoreType.DMA,
            pltpu.SemaphoreType.DMA,
            pltpu.VMEM_SHARED((4 * N,), jnp.float32),
        ],
    )


def _recip_body(acc4_ref, rec_ref):
    a = acc4_ref[0] + acc4_ref[1]
    rec_ref[...] = jnp.where(a > 0.0, 1.0 / jnp.maximum(a, 1e-30), 0.0)


def _recip(acc4):
    return pl.pallas_call(
        _recip_body,
        out_shape=jax.ShapeDtypeStruct((N, 4), jnp.float32),
    )(acc4)


RING = 3
NCH = EPW // C2
NV = NCH + 1
assert NV % RING == 0
assert C1 % C2 == 0


def _sc_agg_body(mnode_hbm, src_hbm, dst_hbm, ae_hbm, rec_hbm, zM_hbm,
                 accM_hbm,
                 as0, as1, as2, ad0, ad1, ad2, ab0, ab1, ab2,
                 ir0, ir1, ir2, gr0, gr1, gr2,
                 mr0, mr1, mr2,
                 sA0, sA1, sA2, sM0, sM1, sM2, sR0, sR1, sR2,
                 sS0, sS1, sS2, accM_sh):
    asrc = [as0, as1, as2]
    adst = [ad0, ad1, ad2]
    abuf = [ab0, ab1, ab2]
    idxr = [ir0, ir1, ir2]
    grc = [gr0, gr1, gr2]
    mrows = [mr0, mr1, mr2]
    semA = [sA0, sA1, sA2]
    semM = [sM0, sM1, sM2]
    semR = [sR0, sR1, sR2]
    semS = [sS0, sS1, sS2]
    c = lax.axis_index("c")
    s = lax.axis_index("s")
    wid = s * NC + c

    @pl.when(s == 0)
    def _():
        pltpu.sync_copy(zM_hbm, accM_sh)
    plsc.subcore_barrier()

    def baseof(j):
        return wid * EPW + j * C2

    def issue_stage_a(j, sl):
        b = baseof(j)
        pltpu.async_copy(src_hbm.at[pl.ds(b, C2)], asrc[sl], semA[sl])
        pltpu.async_copy(dst_hbm.at[pl.ds(b, C2)], adst[sl], semA[sl])
        for i in range(3):
            pltpu.async_copy(ae_hbm.at[pl.ds(i * E + b, C2)],
                             abuf[sl].at[pl.ds(i * C2, C2)], semA[sl])

    def wait_stage_a(j, sl):
        b = baseof(j)
        pltpu.make_async_copy(src_hbm.at[pl.ds(b, C2)], asrc[sl], semA[sl]).wait()
        pltpu.make_async_copy(dst_hbm.at[pl.ds(b, C2)], adst[sl], semA[sl]).wait()
        for i in range(3):
            pltpu.make_async_copy(ae_hbm.at[pl.ds(i * E + b, C2)],
                                  abuf[sl].at[pl.ds(i * C2, C2)], semA[sl]).wait()

    def issue_gather(sl):
        pltpu.async_copy(mnode_hbm.at[asrc[sl]], mrows[sl], semM[sl])

        @pl.loop(0, C2 // L)
        def _(g):
            dv = adst[sl][pl.ds(g * L, L)] * 4
            for i in range(3):
                idxr[sl][pl.ds(i * C2 + g * L, L)] = dv + i

        pltpu.async_copy(rec_hbm.at[idxr[sl]], grc[sl], semR[sl])

    def wait_gather(sl):
        pltpu.make_async_copy(mnode_hbm.at[asrc[sl]], mrows[sl], semM[sl]).wait()
        pltpu.make_async_copy(rec_hbm.at[idxr[sl]], grc[sl], semR[sl]).wait()

    def issue_scatter(sl):
        pltpu.async_copy(mrows[sl], accM_sh.at[adst[sl]], semS[sl], add=True)

    def wait_scatter(sl):
        pltpu.make_async_copy(mrows[sl], accM_sh.at[adst[sl]], semS[sl]).wait()

    issue_stage_a(0, 0)
    issue_stage_a(1, 1)
    wait_stage_a(0, 0)
    issue_gather(0)

    @pl.loop(0, NV // RING)
    def _(t):
        for b in range(RING):
            k = t * RING + b

            @pl.when(k + 1 < NCH)
            def _():
                wait_stage_a(k + 1, (b + 1) % RING)
                issue_gather((b + 1) % RING)

            @pl.when(k < NCH)
            def _():
                wait_gather(b)

                @pl.loop(0, C2 // L)
                def _(g):
                    sl0 = pl.ds(g * L, L)
                    sl1 = pl.ds(C2 + g * L, L)
                    sl2 = pl.ds(2 * C2 + g * L, L)
                    w16 = (abuf[b][sl0] * grc[b][sl0]
                           + abuf[b][sl1] * grc[b][sl1]
                           + abuf[b][sl2] * grc[b][sl2]) * (1.0 / 3.0)
                    for j in range(L):
                        wv = jnp.take_along_axis(
                            w16, jnp.full((L,), j, jnp.int32), axis=0)
                        r = g * L + j
                        for cg in range(M // L):
                            sl = pl.ds(cg * L, L)
                            mrows[b][r, sl] = mrows[b][r, sl] * wv

                issue_scatter(b)

            @pl.when(k >= 1)
            def _():
                wait_scatter((b + 2) % RING)

            @pl.when(k + 2 < NCH)
            def _():
                issue_stage_a(k + 2, (b + 2) % RING)

    plsc.subcore_barrier()

    @pl.when(s == 0)
    def _():
        pltpu.sync_copy(accM_sh, accM_hbm.at[c])


@functools.lru_cache(maxsize=None)
def _sc_agg():
    return pl.kernel(
        _sc_agg_body,
        out_type=jax.ShapeDtypeStruct((NC, N, M), jnp.float32),
        mesh=_get_mesh(),
        scratch_types=(
            [pltpu.VMEM((C2,), jnp.int32)] * 6
            + [pltpu.VMEM((3 * C2,), jnp.float32)] * 3
            + [pltpu.VMEM((3 * C2,), jnp.int32)] * 3
            + [pltpu.VMEM((3 * C2,), jnp.float32)] * 3
            + [pltpu.VMEM((C2, M), jnp.float32)] * 3
            + [pltpu.SemaphoreType.DMA] * 12
            + [pltpu.VMEM_SHARED((N, M), jnp.float32)]
        ),
    )


def _combine_body(x_ref, accM_ref, rec_ref, wc1xt, wc1ht, bc1, wc2t, bc2, out_ref):
    sum_m = accM_ref[0] + accM_ref[1]
    hn = sum_m * rec_ref[...][:, 3:4]
    t = jnp.maximum(
        jnp.dot(x_ref[...], wc1xt[...], preferred_element_type=jnp.float32)
        + jnp.dot(hn, wc1ht[...], preferred_element_type=jnp.float32)
        + bc1[...], 0.0)
    out_ref[...] = jnp.dot(t, wc2t[...], preferred_element_type=jnp.float32) + bc2[...]


def _combine(x, accM, rec, wc1xt, wc1ht, bc1, wc2t, bc2):
    BN = 1000
    grid = N // BN
    return pl.pallas_call(
        _combine_body,
        grid=(grid,),
        in_specs=[
            pl.BlockSpec((BN, D), lambda i: (i, 0)),
            pl.BlockSpec((NC, BN, M), lambda i: (0, i, 0)),
            pl.BlockSpec((BN, 4), lambda i: (i, 0)),
            pl.BlockSpec((D, O), lambda i: (0, 0)),
            pl.BlockSpec((M, O), lambda i: (0, 0)),
            pl.BlockSpec((1, O), lambda i: (0, 0)),
            pl.BlockSpec((O, O), lambda i: (0, 0)),
            pl.BlockSpec((1, O), lambda i: (0, 0)),
        ],
        out_specs=pl.BlockSpec((BN, O), lambda i: (i, 0)),
        out_shape=jax.ShapeDtypeStruct((N, O), jnp.float32),
    )(x, accM, rec, wc1xt, wc1ht, bc1, wc2t, bc2)


@jax.jit
def kernel(x, edge_index, Wm1, bm1, Wm2, bm2, Wm3, bm3,
           Wa1, ba1, Wa2, ba2, Wa3, ba3, Wc1, bc1, Wc2, bc2):
    src = edge_index[0]
    dst = edge_index[1]

    zcol = jnp.zeros((D, 1), jnp.float32)
    wsrc = jnp.concatenate([Wa1[:, :D].T, Wa2[:, :D].T, Wa3[:, :D].T, zcol], axis=1)
    wdst = jnp.concatenate([Wa1[:, D:].T, Wa2[:, D:].T, Wa3[:, D:].T, zcol], axis=1)
    bsrc = jnp.concatenate([ba1, ba2, ba3, jnp.zeros((1,), jnp.float32)]).reshape(1, 4)

    m_node, ssrc, sdst = _node_precompute(
        x, Wm1.T, Wm2.T, Wm3.T,
        bm1.reshape(1, M), bm2.reshape(1, M), bm3.reshape(1, M),
        wsrc, wdst, bsrc)

    z4 = jnp.zeros((4 * N,), jnp.float32)
    ae, acc4 = _sc_att()(ssrc.reshape(4 * N), sdst.reshape(4 * N), src, dst, z4)

    rec = _recip(acc4.reshape(NC, N, 4))

    zM = jnp.zeros((N, M), jnp.float32)
    accM = _sc_agg()(m_node, src, dst, ae, rec.reshape(4 * N), zM)

    return _combine(x, accM, rec,
                    Wc1[:, :D].T, Wc1[:, D:].T, bc1.reshape(1, O),
                    Wc2.T, bc2.reshape(1, O))
